# baseline (speedup 1.0000x reference)
import functools

import jax
import jax.numpy as jnp
from jax import lax
from jax.experimental import pallas as pl
from jax.experimental.pallas import tpu as pltpu
from jax.experimental.pallas import tpu_sc as plsc

_VOCAB = 1000
_NC = 2
_NS = 16
_NW = _NC * _NS
_LANES = 16
_NBUF = 3
_RB = 2


def _make_sc_embed(B, S, D, SX):
    BW = B // _NW
    NV = D // _LANES
    CH = _RB * S
    NCH = BW // _RB
    NG = CH // 128
    GT = CH - NG * 128

    mesh = plsc.VectorSubcoreMesh(core_axis_name="c", subcore_axis_name="s")

    scratch = (
        [pltpu.VMEM((S * D,), jnp.float32)]
        + [pltpu.VMEM((BW, SX), jnp.int32)]
        + [pltpu.VMEM((_RB, 1, S), jnp.int32)] * _NBUF
        + [pltpu.VMEM((_RB, S, D), jnp.float32)] * _NBUF
        + [pltpu.SemaphoreType.DMA]
        + [pltpu.SemaphoreType.DMA] * (2 * _NBUF)
    )

    @functools.partial(
        pl.kernel,
        mesh=mesh,
        out_type=jax.ShapeDtypeStruct((B, S, D), jnp.float32),
        scratch_types=scratch,
    )
    def sc_embed(x_hbm, emb_hbm, pos_hbm, out_hbm, pos_v, xv, *bufs_sems):
        idx = bufs_sems[:_NBUF]
        rows = bufs_sems[_NBUF:2 * _NBUF]
        xsem = bufs_sems[2 * _NBUF]
        gsem = bufs_sems[2 * _NBUF + 1:3 * _NBUF + 1]
        ssem = bufs_sems[3 * _NBUF + 1:]
        cid = lax.axis_index("c")
        sid = lax.axis_index("s")
        w = sid * _NC + cid
        b0 = w * BW
        r0 = w * (BW * S)

        pltpu.sync_copy(pos_hbm, pos_v)
        pltpu.make_async_copy(x_hbm.at[pl.ds(b0, BW), :], xv, xsem).start()

        gsplat = jnp.full((_LANES,), _VOCAB, jnp.int32)

        def build_idx(c, p):
            c = jnp.asarray(c, jnp.int32)
            for r in range(_RB):
                row = c * _RB + r
                idx[p][r, 0, pl.ds(0, _LANES)] = gsplat
                for k in range(SX // _LANES):
                    vals = xv[row, pl.ds(k * _LANES, _LANES)]
                    idx[p][r, 0, pl.ds(1 + k * _LANES, _LANES)] = vals

        def gathers_start(p):
            for r in range(_RB):
                pltpu.async_copy(
                    emb_hbm.at[idx[p].at[r, 0]], rows[p].at[r], gsem[p]
                )

        def gathers_wait(p):
            for r in range(_RB):
                pltpu.make_async_copy(
                    emb_hbm.at[idx[p].at[r, 0]], rows[p].at[r], gsem[p]
                ).wait()

        def store(c, p):
            c = jnp.asarray(c, jnp.int32)
            return pltpu.make_async_copy(
                rows[p], out_hbm.at[pl.ds(b0 + c * _RB, _RB), :, :], ssem[p]
            )

        def add_pos(p):
            def add_j(j, carry):
                pv = [pos_v[pl.ds(j * D + v * _LANES, _LANES)] for v in range(NV)]
                for r in range(_RB):
                    for v in range(NV):
                        sl = pl.ds(v * _LANES, _LANES)
                        rows[p][r, j, sl] = rows[p][r, j, sl] + pv[v]
                return carry

            lax.fori_loop(0, S, add_j, 0)

        pltpu.make_async_copy(x_hbm.at[pl.ds(b0, BW), :], xv, xsem).wait()
        build_idx(0, 0)
        gathers_start(0)
        build_idx(1, 1)

        def step(c, p):
            @pl.when(c + 1 < NCH)
            def _():
                pn = (p + 1) % _NBUF

                @pl.when(c + 1 >= _NBUF)
                def _():
                    store(c + 1 - _NBUF, pn).wait()

                gathers_start(pn)

            @pl.when(c + 2 < NCH)
            def _():
                build_idx(c + 2, (p + 2) % _NBUF)

            gathers_wait(p)
            add_pos(p)
            store(c, p).start()

        def body(i, carry):
            c = i * _NBUF
            for q in range(_NBUF):
                step(c + q, q)
            return carry

        lax.fori_loop(0, NCH // _NBUF, body, 0)
        for q in range(NCH % _NBUF):
            cq = NCH - (NCH % _NBUF) + q
            step(jnp.int32(cq), cq % _NBUF)

        for c in range(max(0, NCH - _NBUF), NCH):
            store(jnp.int32(c), c % _NBUF).wait()

    return sc_embed


def kernel(x, emb_table, pos_table, pos_ids):
    B, SX = x.shape
    Sg = SX + 1
    D = emb_table.shape[1]
    pos_flat = jnp.take(pos_table, pos_ids[0], axis=0).astype(jnp.float32).reshape(Sg * D)
    return _make_sc_embed(B, Sg, D, SX)(x.astype(jnp.int32), emb_table, pos_flat)


# speedup vs baseline: 1.1607x; 1.1607x over previous
import functools

import jax
import jax.numpy as jnp
from jax import lax
from jax.experimental import pallas as pl
from jax.experimental.pallas import tpu as pltpu
from jax.experimental.pallas import tpu_sc as plsc

_VOCAB = 1000
_NC = 2
_NS = 16
_NW = _NC * _NS
_LANES = 16
_NBUF = 4


def _make_sc_embed(B, S, D):
    BW = B // _NW
    NV = D // _LANES

    mesh = plsc.VectorSubcoreMesh(core_axis_name="c", subcore_axis_name="s")

    scratch = (
        [pltpu.VMEM((S * D,), jnp.float32)]
        + [pltpu.VMEM((BW,), jnp.int32)] * _NBUF
        + [pltpu.VMEM((BW, 1, D), jnp.float32)] * _NBUF
        + [pltpu.SemaphoreType.DMA] * (3 * _NBUF)
    )

    @functools.partial(
        pl.kernel,
        mesh=mesh,
        out_type=jax.ShapeDtypeStruct((B, S, D), jnp.float32),
        scratch_types=scratch,
    )
    def sc_embed(xgT_hbm, emb_hbm, pos_hbm, out_hbm, pos_v, *bufs_sems):
        idx = bufs_sems[:_NBUF]
        rows = bufs_sems[_NBUF:2 * _NBUF]
        isem = bufs_sems[2 * _NBUF:3 * _NBUF]
        gsem = bufs_sems[3 * _NBUF:4 * _NBUF]
        ssem = bufs_sems[4 * _NBUF:]
        cid = lax.axis_index("c")
        sid = lax.axis_index("s")
        w = sid * _NC + cid
        b0 = w * BW

        pltpu.sync_copy(pos_hbm, pos_v)

        def idx_copy(j, p):
            j = jnp.asarray(j, jnp.int32)
            return pltpu.make_async_copy(
                xgT_hbm.at[pl.ds(j * B + b0, BW)], idx[p], isem[p]
            )

        def gather(j, p):
            return pltpu.make_async_copy(emb_hbm.at[idx[p]], rows[p].at[:, 0], gsem[p])

        def store(j, p):
            j = jnp.asarray(j, jnp.int32)
            return pltpu.make_async_copy(
                rows[p], out_hbm.at[pl.ds(b0, BW), pl.ds(j, 1), :], ssem[p]
            )

        def add_pos(j, p):
            j = jnp.asarray(j, jnp.int32)
            pv = [pos_v[pl.ds(j * D + v * _LANES, _LANES)] for v in range(NV)]

            def add_row(i, pv):
                for v in range(NV):
                    sl = pl.ds(v * _LANES, _LANES)
                    rows[p][i, 0, sl] = rows[p][i, 0, sl] + pv[v]
                return pv

            lax.fori_loop(0, BW, add_row, pv, unroll=2)

        idx_copy(0, 0).start()
        idx_copy(1, 1).start()
        idx_copy(0, 0).wait()
        gather(0, 0).start()
        idx_copy(1, 1).wait()
        gather(1, 1).start()
        idx_copy(2, 2).start()

        def step(j, p):
            @pl.when(j + 2 < S)
            def _():
                pn = (p + 2) % _NBUF
                idx_copy(j + 2, pn).wait()

                @pl.when(j >= 2)
                def _():
                    store(j - 2, pn).wait()

                gather(j + 2, pn).start()

            @pl.when(j + 3 < S)
            def _():
                pn = (p + 3) % _NBUF
                idx_copy(j + 3, pn).start()

            gather(j, p).wait()
            add_pos(j, p)
            store(j, p).start()

        def body(i, carry):
            j = i * _NBUF
            for q in range(_NBUF):
                step(j + q, q)
            return carry

        lax.fori_loop(0, S // _NBUF, body, 0)
        for q in range(S % _NBUF):
            jt = jnp.int32(S - (S % _NBUF) + q)
            step(jt, q)

        for j in range(S - _NBUF, S):
            store(jnp.int32(j), j % _NBUF).wait()

    return sc_embed


def kernel(x, emb_table, pos_table, pos_ids):
    B, S = x.shape
    Sg = S + 1
    D = emb_table.shape[1]
    xgT = jnp.concatenate(
        [jnp.full((1, B), _VOCAB, x.dtype), x.T.astype(jnp.int32)], axis=0
    ).reshape(Sg * B)
    pos_eff = jnp.take(pos_table, pos_ids[0], axis=0).astype(jnp.float32)
    pos_flat = pos_eff.reshape(Sg * D)
    return _make_sc_embed(B, Sg, D)(xgT, emb_table, pos_flat)
